# baseline (speedup 1.0000x reference)
import functools

import jax
import jax.numpy as jnp
from jax import lax
from jax.experimental import pallas as pl
from jax.experimental.pallas import tpu as pltpu
from jax.experimental.pallas import tpu_sc as plsc

_B, _F, _V, _D = 16384, 26, 100000, 13
_NC, _NS, _L = 2, 16, 16
_FH = _F // _NC
_CB = _B // _NS
_CH = 128
_NCH = _CB // _CH

_E = 1299968
_SH = _E + _CH
_TW = 99968
_REM = _E - _FH * _TW
_SA = 3200
_SC_ = 768
_STCH = (_SA,) * 31 + (_SC_,)


def _sc_body(sf_hbm, dn_hbm, tbl_hbm, wb_hbm, p0_hbm, p1_hbm,
             shared, idx_v, g_v, dn_v, wb_v, out_v, sem):
  cid = lax.axis_index("c")
  sid = lax.axis_index("s")
  base = sid * _CB

  @pl.when(sid < _FH)
  def _stage_main():
    src0 = cid * _E + sid * _TW
    dst0 = sid * _TW
    n = len(_STCH)
    bufs = [g_v.at[pl.ds(0, _SA)] if h % 2 == 0 else g_v.at[pl.ds(_SA, _SA)]
            for h in range(n)]
    bufs[n - 1] = g_v.at[pl.ds(2 * _SA, _SC_)]
    cps = []
    off = 0
    offs = []
    for h in range(n):
      offs.append(off)
      cps.append(pltpu.async_copy(
          tbl_hbm.at[pl.ds(src0 + off, _STCH[h])], bufs[h], sem))
      off += _STCH[h]
      if h >= 1:
        cps[h - 1].wait()
        pltpu.sync_copy(bufs[h - 1],
                        shared.at[pl.ds(dst0 + offs[h - 1], _STCH[h - 1])])
    cps[n - 1].wait()
    pltpu.sync_copy(bufs[n - 1],
                    shared.at[pl.ds(dst0 + offs[n - 1], _STCH[n - 1])])

  @pl.when(sid >= _FH)
  def _stage_rem():
    off = _FH * _TW + (sid - _FH) * _CH
    stg_t = g_v.at[pl.ds(2 * _SA + _SC_, _CH)]
    pltpu.sync_copy(tbl_hbm.at[pl.ds(cid * _E + off, _CH)], stg_t)
    pltpu.sync_copy(stg_t, shared.at[pl.ds(off, _CH)])

  @pl.when(sid == _NS - 1)
  def _stage_tail():
    tail_src = (cid + 1) * _FH * _V - _CH
    stg_t = g_v.at[pl.ds(2 * _SA + _SC_, _CH)]
    pltpu.sync_copy(tbl_hbm.at[pl.ds(tail_src, _CH)], stg_t)
    pltpu.sync_copy(stg_t, shared.at[pl.ds(_E, _CH)])

  pltpu.sync_copy(
      sf_hbm.at[pl.ds(cid * _FH, _FH), pl.ds(sid * _NCH, _NCH), :], idx_v)
  pltpu.sync_copy(dn_hbm.at[:, pl.ds(base, _CB)], dn_v)
  pltpu.sync_copy(wb_hbm, wb_v)

  tshift = jnp.int32(96) - jnp.int32(32) * cid.astype(jnp.int32)
  foff0 = jnp.int32(32) * cid.astype(jnp.int32)

  def add_off(c, carry):
    for f in range(_FH):
      off = jnp.int32(f * _V) + foff0
      for l in range(_CH // _L):
        s = pl.ds(l * _L, _L)
        v = idx_v[f, c, s] + off
        if f == _FH - 1:
          v = v + jnp.where(v >= jnp.int32(_E), tshift, jnp.int32(0))
        idx_v[f, c, s] = v
    return carry
  lax.fori_loop(0, _NCH, add_off, 0)

  plsc.subcore_barrier()

  copies = []
  for f in range(_FH):
    for c in range(_NCH):
      copies.append(
          pltpu.async_copy(shared.at[idx_v.at[f, c]],
                           g_v.at[pl.ds((f * _NCH + c) * _CH, _CH)], sem))

  wvec = wb_v[pl.ds(0, _L)]
  bias = wvec[_D]

  def init_c(c, carry):
    for l in range(_CH // _L):
      s2 = pl.ds(c * _CH + l * _L, _L)
      acc = jnp.full((_L,), bias, jnp.float32)
      for d in range(_D):
        acc = acc + wvec[d] * dn_v[d, s2]
      out_v[s2] = acc
    return carry

  @pl.when(cid == 0)
  def _dense():
    lax.fori_loop(0, _NCH, init_c, 0)

  @pl.when(cid != 0)
  def _zero():
    def zero_c(c, carry):
      for l in range(_CH // _L):
        s2 = pl.ds(c * _CH + l * _L, _L)
        out_v[s2] = jnp.zeros((_L,), jnp.float32)
      return carry
    lax.fori_loop(0, _NCH, zero_c, 0)

  for cp in copies:
    cp.wait()

  def reduce_c(c, carry):
    for l in range(_CH // _L):
      s = pl.ds(l * _L, _L)
      s2 = pl.ds(c * _CH + l * _L, _L)
      acc = out_v[s2]
      for f in range(_FH):
        acc = acc + g_v[pl.ds(f * _NCH * _CH + c * _CH + l * _L, _L)]
      out_v[s2] = acc
    return carry
  lax.fori_loop(0, _NCH, reduce_c, 0)

  @pl.when(cid == 0)
  def _w0():
    pltpu.sync_copy(out_v, p0_hbm.at[pl.ds(base, _CB)])

  @pl.when(cid != 0)
  def _w1():
    pltpu.sync_copy(out_v, p1_hbm.at[pl.ds(base, _CB)])


def _add_body(a_ref, b_ref, o_ref):
  o_ref[...] = a_ref[...] + b_ref[...]


@jax.jit
def _run(sf_r, dn_t, tbl, wb):
  mesh = plsc.VectorSubcoreMesh(core_axis_name="c", subcore_axis_name="s")
  p0, p1 = pl.kernel(
      _sc_body,
      out_type=(jax.ShapeDtypeStruct((_B,), jnp.float32),
                jax.ShapeDtypeStruct((_B,), jnp.float32)),
      mesh=mesh,
      scratch_types=[
          pltpu.VMEM_SHARED((_SH,), jnp.float32),
          pltpu.VMEM((_FH, _NCH, _CH), jnp.int32),
          pltpu.VMEM((_FH * _NCH * _CH,), jnp.float32),
          pltpu.VMEM((_D, _CB), jnp.float32),
          pltpu.VMEM((_L,), jnp.float32),
          pltpu.VMEM((_CB,), jnp.float32),
          pltpu.SemaphoreType.DMA,
      ],
  )(sf_r, dn_t, tbl, wb)
  out2d = pl.pallas_call(
      _add_body,
      out_shape=jax.ShapeDtypeStruct((_B // 128, 128), jnp.float32),
  )(p0.reshape(_B // 128, 128), p1.reshape(_B // 128, 128))
  return out2d.reshape(_B)


def kernel(sparse_feat, dense_feat, W_sparse, w_dense, b):
  sf_r = sparse_feat.astype(jnp.int32).T.reshape(_F, _B // _CH, _CH)
  dn_t = dense_feat.astype(jnp.float32).T
  tbl = W_sparse.reshape(_F * _V)
  wb = jnp.concatenate(
      [w_dense, b, jnp.zeros((_L - _D - 1,), jnp.float32)])
  return _run(sf_r, dn_t, tbl, wb)


# speedup vs baseline: 1.4269x; 1.4269x over previous
import functools

import jax
import jax.numpy as jnp
from jax import lax
from jax.experimental import pallas as pl
from jax.experimental.pallas import tpu as pltpu
from jax.experimental.pallas import tpu_sc as plsc

_B, _F, _V, _D = 16384, 26, 100000, 13
_NC, _NS, _L = 2, 16, 16
_FH = _F // _NC
_CB = _B // _NS
_CH = 128
_NCH = _CB // _CH

_SA = 6400
_SB = 3968
_TAIL = _V - 15 * _SA - _SB
_STCH = (_SA,) * 15 + (_SB,)


def _sc_body(sf_hbm, dn_hbm, tbl_hbm, tails_hbm, wb_hbm, p0_hbm, p1_hbm,
             shared, idx_v, g_v, dn_v, wb_v, out_v, sem):
  cid = lax.axis_index("c")
  sid = lax.axis_index("s")
  base = sid * _CB

  @pl.when(sid < _FH)
  def _stage_main():
    row = cid * _FH + sid
    dst0 = sid * _V
    n = len(_STCH)
    bufs = [g_v.at[pl.ds(0, _SA)] if h % 2 == 0 else g_v.at[pl.ds(_SA, _SA)]
            for h in range(n)]
    bufs[n - 1] = g_v.at[pl.ds(((n - 1) % 2) * _SA, _SB)]
    cps = []
    off = 0
    offs = []
    for h in range(n):
      offs.append(off)
      cps.append(pltpu.async_copy(
          tbl_hbm.at[row, pl.ds(off, _STCH[h])], bufs[h], sem))
      off += _STCH[h]
      if h >= 1:
        cps[h - 1].wait()
        pltpu.sync_copy(bufs[h - 1],
                        shared.at[pl.ds(dst0 + offs[h - 1], _STCH[h - 1])])
    cps[n - 1].wait()
    pltpu.sync_copy(bufs[n - 1],
                    shared.at[pl.ds(dst0 + offs[n - 1], _STCH[n - 1])])
    tl = g_v.at[pl.ds(2 * _SA, _TAIL)]
    pltpu.sync_copy(tails_hbm.at[pl.ds(row * _TAIL, _TAIL)], tl)
    pltpu.sync_copy(tl, shared.at[pl.ds(dst0 + _V - _TAIL, _TAIL)])

  pltpu.sync_copy(
      sf_hbm.at[pl.ds(cid * _FH, _FH), pl.ds(sid * _NCH, _NCH), :], idx_v)
  pltpu.sync_copy(dn_hbm.at[:, pl.ds(base, _CB)], dn_v)
  pltpu.sync_copy(wb_hbm, wb_v)

  def add_off(c, carry):
    for f in range(_FH):
      off = jnp.int32(f * _V)
      for l in range(_CH // _L):
        s = pl.ds(l * _L, _L)
        idx_v[f, c, s] = idx_v[f, c, s] + off
    return carry
  lax.fori_loop(0, _NCH, add_off, 0)

  plsc.subcore_barrier()

  copies = []
  for f in range(_FH):
    for c in range(_NCH):
      copies.append(
          pltpu.async_copy(shared.at[idx_v.at[f, c]],
                           g_v.at[pl.ds((f * _NCH + c) * _CH, _CH)], sem))

  wvec = wb_v[pl.ds(0, _L)]
  bias = wvec[_D]

  def init_c(c, carry):
    for l in range(_CH // _L):
      s2 = pl.ds(c * _CH + l * _L, _L)
      acc = jnp.full((_L,), bias, jnp.float32)
      for d in range(_D):
        acc = acc + wvec[d] * dn_v[d, s2]
      out_v[s2] = acc
    return carry

  @pl.when(cid == 0)
  def _dense():
    lax.fori_loop(0, _NCH, init_c, 0)

  @pl.when(cid != 0)
  def _zero():
    def zero_c(c, carry):
      for l in range(_CH // _L):
        s2 = pl.ds(c * _CH + l * _L, _L)
        out_v[s2] = jnp.zeros((_L,), jnp.float32)
      return carry
    lax.fori_loop(0, _NCH, zero_c, 0)

  for cp in copies:
    cp.wait()

  def reduce_c(c, carry):
    for l in range(_CH // _L):
      s2 = pl.ds(c * _CH + l * _L, _L)
      acc = out_v[s2]
      for f in range(_FH):
        acc = acc + g_v[pl.ds(f * _NCH * _CH + c * _CH + l * _L, _L)]
      out_v[s2] = acc
    return carry
  lax.fori_loop(0, _NCH, reduce_c, 0)

  @pl.when(cid == 0)
  def _w0():
    pltpu.sync_copy(out_v, p0_hbm.at[pl.ds(base, _CB)])

  @pl.when(cid != 0)
  def _w1():
    pltpu.sync_copy(out_v, p1_hbm.at[pl.ds(base, _CB)])


def _add_body(a_ref, b_ref, o_ref):
  o_ref[...] = a_ref[...] + b_ref[...]


@jax.jit
def _run(sf_r, dn_t, tbl, tails, wb):
  mesh = plsc.VectorSubcoreMesh(core_axis_name="c", subcore_axis_name="s")
  p0, p1 = pl.kernel(
      _sc_body,
      out_type=(jax.ShapeDtypeStruct((_B,), jnp.float32),
                jax.ShapeDtypeStruct((_B,), jnp.float32)),
      mesh=mesh,
      scratch_types=[
          pltpu.VMEM_SHARED((_FH * _V,), jnp.float32),
          pltpu.VMEM((_FH, _NCH, _CH), jnp.int32),
          pltpu.VMEM((_FH * _NCH * _CH,), jnp.float32),
          pltpu.VMEM((_D, _CB), jnp.float32),
          pltpu.VMEM((_L,), jnp.float32),
          pltpu.VMEM((_CB,), jnp.float32),
          pltpu.SemaphoreType.DMA,
      ],
  )(sf_r, dn_t, tbl, tails, wb)
  out2d = pl.pallas_call(
      _add_body,
      out_shape=jax.ShapeDtypeStruct((_B // 128, 128), jnp.float32),
  )(p0.reshape(_B // 128, 128), p1.reshape(_B // 128, 128))
  return out2d.reshape(_B)


def kernel(sparse_feat, dense_feat, W_sparse, w_dense, b):
  sf_r = sparse_feat.astype(jnp.int32).T.reshape(_F, _B // _CH, _CH)
  dn_t = dense_feat.astype(jnp.float32).T
  wb = jnp.concatenate(
      [w_dense, b, jnp.zeros((_L - _D - 1,), jnp.float32)])
  tails = W_sparse[:, _V - _TAIL:].reshape(_F * _TAIL)
  return _run(sf_r, dn_t, W_sparse, tails, wb)


# speedup vs baseline: 1.4289x; 1.0014x over previous
import functools

import jax
import jax.numpy as jnp
from jax import lax
from jax.experimental import pallas as pl
from jax.experimental.pallas import tpu as pltpu
from jax.experimental.pallas import tpu_sc as plsc

_B, _F, _V, _D = 16384, 26, 100000, 13
_NC, _NS, _L = 2, 16, 16
_FH = _F // _NC
_CB = _B // _NS
_CH = 128
_NCH = _CB // _CH

_SA = 6400
_SB = 3968
_TAIL = _V - 15 * _SA - _SB
_STCH = (_SA,) * 15 + (_SB,)


def _sc_body(sf_hbm, dn_hbm, tbl_hbm, tails_hbm, wb_hbm, p0_hbm, p1_hbm,
             shared, idx_v, g_v, dn_v, wb_v, out_v, sem):
  cid = lax.axis_index("c")
  sid = lax.axis_index("s")
  base = sid * _CB

  @pl.when(sid < _FH)
  def _stage_main():
    row = cid * _FH + sid
    dst0 = sid * _V
    n = len(_STCH)
    bufs = [g_v.at[pl.ds(0, _SA)] if h % 2 == 0 else g_v.at[pl.ds(_SA, _SA)]
            for h in range(n)]
    bufs[n - 1] = g_v.at[pl.ds(((n - 1) % 2) * _SA, _SB)]
    cps = []
    off = 0
    offs = []
    for h in range(n):
      offs.append(off)
      cps.append(pltpu.async_copy(
          tbl_hbm.at[row, pl.ds(off, _STCH[h])], bufs[h], sem))
      off += _STCH[h]
      if h >= 1:
        cps[h - 1].wait()
        pltpu.sync_copy(bufs[h - 1],
                        shared.at[pl.ds(dst0 + offs[h - 1], _STCH[h - 1])])
    cps[n - 1].wait()
    pltpu.sync_copy(bufs[n - 1],
                    shared.at[pl.ds(dst0 + offs[n - 1], _STCH[n - 1])])
    tl = g_v.at[pl.ds(2 * _SA, _TAIL)]
    pltpu.sync_copy(tails_hbm.at[pl.ds(row * _TAIL, _TAIL)], tl)
    pltpu.sync_copy(tl, shared.at[pl.ds(dst0 + _V - _TAIL, _TAIL)])

  pltpu.sync_copy(
      sf_hbm.at[pl.ds(cid * _FH, _FH), pl.ds(sid * _NCH, _NCH), :], idx_v)
  pltpu.sync_copy(dn_hbm.at[:, pl.ds(base, _CB)], dn_v)
  pltpu.sync_copy(wb_hbm, wb_v)

  def add_off(c, carry):
    for f in range(_FH):
      off = jnp.int32(f * _V)
      for l in range(_CH // _L):
        s = pl.ds(l * _L, _L)
        idx_v[f, c, s] = idx_v[f, c, s] + off
    return carry
  lax.fori_loop(0, _NCH, add_off, 0)

  with jax.named_scope("ph_barrier"):
    plsc.subcore_barrier()

  copies = []
  for f in range(_FH):
    for c in range(_NCH):
      copies.append(
          pltpu.async_copy(shared.at[idx_v.at[f, c]],
                           g_v.at[pl.ds((f * _NCH + c) * _CH, _CH)], sem))

  wvec = wb_v[pl.ds(0, _L)]
  bias = wvec[_D]

  def init_c(c, carry):
    for l in range(_CH // _L):
      s2 = pl.ds(c * _CH + l * _L, _L)
      acc = jnp.full((_L,), bias, jnp.float32)
      for d in range(_D):
        acc = acc + wvec[d] * dn_v[d, s2]
      out_v[s2] = acc
    return carry

  @pl.when(cid == 0)
  def _dense():
    lax.fori_loop(0, _NCH, init_c, 0)

  @pl.when(cid != 0)
  def _zero():
    def zero_c(c, carry):
      for l in range(_CH // _L):
        s2 = pl.ds(c * _CH + l * _L, _L)
        out_v[s2] = jnp.zeros((_L,), jnp.float32)
      return carry
    lax.fori_loop(0, _NCH, zero_c, 0)

  with jax.named_scope("ph_drain"):
    for cp in copies:
      cp.wait()

  def reduce_c(c, carry):
    for l in range(_CH // _L):
      s2 = pl.ds(c * _CH + l * _L, _L)
      acc = out_v[s2]
      for f in range(_FH):
        acc = acc + g_v[pl.ds(f * _NCH * _CH + c * _CH + l * _L, _L)]
      out_v[s2] = acc
    return carry
  with jax.named_scope("ph_reduce"):
    lax.fori_loop(0, _NCH, reduce_c, 0)

  @pl.when(cid == 0)
  def _w0():
    pltpu.sync_copy(out_v, p0_hbm.at[pl.ds(base, _CB)])

  @pl.when(cid != 0)
  def _w1():
    pltpu.sync_copy(out_v, p1_hbm.at[pl.ds(base, _CB)])


def _add_body(a_ref, b_ref, o_ref):
  o_ref[...] = a_ref[...] + b_ref[...]


@jax.jit
def _run(sf_r, dn_t, tbl, tails, wb):
  mesh = plsc.VectorSubcoreMesh(core_axis_name="c", subcore_axis_name="s")
  p0, p1 = pl.kernel(
      _sc_body,
      out_type=(jax.ShapeDtypeStruct((_B,), jnp.float32),
                jax.ShapeDtypeStruct((_B,), jnp.float32)),
      mesh=mesh,
      scratch_types=[
          pltpu.VMEM_SHARED((_FH * _V,), jnp.float32),
          pltpu.VMEM((_FH, _NCH, _CH), jnp.int32),
          pltpu.VMEM((_FH * _NCH * _CH,), jnp.float32),
          pltpu.VMEM((_D, _CB), jnp.float32),
          pltpu.VMEM((_L,), jnp.float32),
          pltpu.VMEM((_CB,), jnp.float32),
          pltpu.SemaphoreType.DMA,
      ],
  )(sf_r, dn_t, tbl, tails, wb)
  out2d = pl.pallas_call(
      _add_body,
      out_shape=jax.ShapeDtypeStruct((_B // 128, 128), jnp.float32),
  )(p0.reshape(_B // 128, 128), p1.reshape(_B // 128, 128))
  return out2d.reshape(_B)


def kernel(sparse_feat, dense_feat, W_sparse, w_dense, b):
  sf_r = sparse_feat.astype(jnp.int32).T.reshape(_F, _B // _CH, _CH)
  dn_t = dense_feat.astype(jnp.float32).T
  wb = jnp.concatenate(
      [w_dense, b, jnp.zeros((_L - _D - 1,), jnp.float32)])
  tails = W_sparse[:, _V - _TAIL:].reshape(_F * _TAIL)
  return _run(sf_r, dn_t, W_sparse, tails, wb)
